# Initial kernel scaffold; baseline (speedup 1.0000x reference)
import jax
import jax.numpy as jnp
from jax.experimental import pallas as pl


def kernel(p, time, params):
    raise NotImplementedError("write your pallas kernel here")


# speedup vs baseline: 549.5742x; 549.5742x over previous
import functools

import jax
import jax.numpy as jnp
import numpy as np
from jax.experimental import pallas as pl

_EPS = 1e-3
_B, _N, _F, _T, _H = 2, 384, 128, 8, 64
_INV_DEG = 1.0 / (_N - 1)


def _fold(layer):
    s = layer["gamma"] * (1.0 / np.sqrt(1.0 + _EPS))
    w = layer["W"] * s[:, None]
    b = (layer["b"] + layer["beta"] @ layer["W"])[None, :]
    return w, b


def _gnn_body(time_ref, p_ref, *refs):
    out_ref = refs[-1]
    (tm_w1, tm_b1, tm_w2, tm_b2,
     pre_w1, pre_b1, pre_w2, pre_b2,
     c1p_wx, c1p_wt, c1p_b1, c1p_w2, c1p_b2,
     c1u_wx, c1u_wa, c1u_wt, c1u_b1, c1u_w2, c1u_b2,
     c2p_wx, c2p_wt, c2p_b1, c2p_w2, c2p_b2,
     c2u_wx, c2u_wa, c2u_wt, c2u_b1, c2u_w2, c2u_b2,
     post_w1, post_b1, post_w2, post_b2,
     lg_w, lg_b) = [r[...] for r in refs[:-1]]

    dot = functools.partial(jnp.dot, precision=jax.lax.Precision.HIGHEST,
                            preferred_element_type=jnp.float32)
    gelu = jax.nn.gelu

    t = gelu(dot(time_ref[0], tm_w1) + tm_b1)
    t = gelu(dot(t, tm_w2) + tm_b2)

    x = p_ref[0]
    x = gelu(dot(x, pre_w1) + pre_b1)
    x = gelu(dot(x, pre_w2) + pre_b2)

    for (p_wx, p_wt, p_b1, p_w2, p_b2,
         u_wx, u_wa, u_wt, u_b1, u_w2, u_b2) in (
            (c1p_wx, c1p_wt, c1p_b1, c1p_w2, c1p_b2,
             c1u_wx, c1u_wa, c1u_wt, c1u_b1, c1u_w2, c1u_b2),
            (c2p_wx, c2p_wt, c2p_b1, c2p_w2, c2p_b2,
             c2u_wx, c2u_wa, c2u_wt, c2u_b1, c2u_w2, c2u_b2)):
        g = gelu(dot(x, p_wx) + (dot(t, p_wt) + p_b1))
        g = gelu(dot(g, p_w2) + p_b2)
        s = jnp.sum(g, axis=0, keepdims=True)
        agg = (s - g) * _INV_DEG
        u = gelu(dot(x, u_wx) + dot(agg, u_wa) + (dot(t, u_wt) + u_b1))
        u = gelu(dot(u, u_w2) + u_b2)
        x = x + u

    x = gelu(dot(x, post_w1) + post_b1)
    x = gelu(dot(x, post_w2) + post_b2)
    out_ref[0] = dot(x, lg_w) + lg_b


def kernel(p, time, params):
    tm_w1, tm_b1 = _fold(params["time_mlp"][0])
    tm_w2, tm_b2 = _fold(params["time_mlp"][1])
    pre_w1, pre_b1 = _fold(params["pre"][0])
    pre_w2, pre_b2 = _fold(params["pre"][1])

    def conv_weights(prep_key, upd_key):
        p_w1, p_b1 = _fold(params[prep_key][0])
        p_w2, p_b2 = _fold(params[prep_key][1])
        u_w1, u_b1 = _fold(params[upd_key][0])
        u_w2, u_b2 = _fold(params[upd_key][1])
        return (p_w1[:_H], p_w1[_H:], p_b1, p_w2, p_b2,
                u_w1[:_H], u_w1[_H:2 * _H], u_w1[2 * _H:], u_b1, u_w2, u_b2)

    c1 = conv_weights("c1_prep", "c1_upd")
    c2 = conv_weights("c2_prep", "c2_upd")
    post_w1, post_b1 = _fold(params["post"][0])
    post_w2, post_b2 = _fold(params["post"][1])

    weights = (tm_w1, tm_b1, tm_w2, tm_b2,
               pre_w1, pre_b1, pre_w2, pre_b2,
               *c1, *c2,
               post_w1, post_b1, post_w2, post_b2,
               params["logits_W"], params["logits_b"][None, :])

    w_specs = [pl.BlockSpec(w.shape, lambda b: (0, 0)) for w in weights]
    return pl.pallas_call(
        _gnn_body,
        grid=(_B,),
        in_specs=[pl.BlockSpec((1, 1, _T), lambda b: (b, 0, 0)),
                  pl.BlockSpec((1, _N, _F), lambda b: (b, 0, 0))] + w_specs,
        out_specs=pl.BlockSpec((1, _N, _F), lambda b: (b, 0, 0)),
        out_shape=jax.ShapeDtypeStruct((_B, _N, _F), jnp.float32),
    )(time[:, None, :], p, *weights)


# speedup vs baseline: 573.7628x; 1.0440x over previous
import functools

import jax
import jax.numpy as jnp
import numpy as np
from jax.experimental import pallas as pl

_EPS = 1e-3
_B, _N, _F, _T, _H = 2, 384, 128, 8, 64
_INV_DEG = 1.0 / (_N - 1)


def _fold(layer):
    s = layer["gamma"] * (1.0 / np.sqrt(1.0 + _EPS))
    w = layer["W"] * s[:, None]
    b = (layer["b"] + layer["beta"] @ layer["W"])[None, :]
    return w, b


def _rows_per_batch(v):
    d = v.shape[1]
    return jnp.concatenate(
        [jnp.broadcast_to(v[i:i + 1], (_N, d)) for i in range(_B)], axis=0)


def _mean_of_others(g):
    d = g.shape[1]
    s = jnp.concatenate(
        [jnp.broadcast_to(jnp.sum(g[i * _N:(i + 1) * _N], axis=0,
                                  keepdims=True), (_N, d))
         for i in range(_B)], axis=0)
    return (s - g) * _INV_DEG


def _gnn_body(time_ref, p_ref, *refs):
    out_ref = refs[-1]
    (tm_w1, tm_b1, tm_w2, tm_b2,
     pre_w1, pre_b1, pre_w2, pre_b2,
     c1p_wx, c1p_wt, c1p_b1, c1p_w2, c1p_b2,
     c1u_wx, c1u_wa, c1u_wt, c1u_b1, c1u_w2, c1u_b2,
     c2p_wx, c2p_wt, c2p_b1, c2p_w2, c2p_b2,
     c2u_wx, c2u_wa, c2u_wt, c2u_b1, c2u_w2, c2u_b2,
     post_w1, post_b1, post_w2, post_b2,
     lg_w, lg_b) = [r[...] for r in refs[:-1]]

    dot = functools.partial(jnp.dot, precision=jax.lax.Precision.HIGHEST,
                            preferred_element_type=jnp.float32)
    gelu = jax.nn.gelu

    t = gelu(dot(time_ref[...], tm_w1) + tm_b1)
    t = gelu(dot(t, tm_w2) + tm_b2)

    x = p_ref[...]
    x = gelu(dot(x, pre_w1) + pre_b1)
    x = gelu(dot(x, pre_w2) + pre_b2)

    for (p_wx, p_wt, p_b1, p_w2, p_b2,
         u_wx, u_wa, u_wt, u_b1, u_w2, u_b2) in (
            (c1p_wx, c1p_wt, c1p_b1, c1p_w2, c1p_b2,
             c1u_wx, c1u_wa, c1u_wt, c1u_b1, c1u_w2, c1u_b2),
            (c2p_wx, c2p_wt, c2p_b1, c2p_w2, c2p_b2,
             c2u_wx, c2u_wa, c2u_wt, c2u_b1, c2u_w2, c2u_b2)):
        g = gelu(dot(x, p_wx) + _rows_per_batch(dot(t, p_wt) + p_b1))
        g = gelu(dot(g, p_w2) + p_b2)
        agg = _mean_of_others(g)
        u = gelu(dot(x, u_wx) + dot(agg, u_wa)
                 + _rows_per_batch(dot(t, u_wt) + u_b1))
        u = gelu(dot(u, u_w2) + u_b2)
        x = x + u

    x = gelu(dot(x, post_w1) + post_b1)
    x = gelu(dot(x, post_w2) + post_b2)
    out_ref[...] = dot(x, lg_w) + lg_b


def kernel(p, time, params):
    tm_w1, tm_b1 = _fold(params["time_mlp"][0])
    tm_w2, tm_b2 = _fold(params["time_mlp"][1])
    pre_w1, pre_b1 = _fold(params["pre"][0])
    pre_w2, pre_b2 = _fold(params["pre"][1])

    def conv_weights(prep_key, upd_key):
        p_w1, p_b1 = _fold(params[prep_key][0])
        p_w2, p_b2 = _fold(params[prep_key][1])
        u_w1, u_b1 = _fold(params[upd_key][0])
        u_w2, u_b2 = _fold(params[upd_key][1])
        return (p_w1[:_H], p_w1[_H:], p_b1, p_w2, p_b2,
                u_w1[:_H], u_w1[_H:2 * _H], u_w1[2 * _H:], u_b1, u_w2, u_b2)

    c1 = conv_weights("c1_prep", "c1_upd")
    c2 = conv_weights("c2_prep", "c2_upd")
    post_w1, post_b1 = _fold(params["post"][0])
    post_w2, post_b2 = _fold(params["post"][1])

    weights = (tm_w1, tm_b1, tm_w2, tm_b2,
               pre_w1, pre_b1, pre_w2, pre_b2,
               *c1, *c2,
               post_w1, post_b1, post_w2, post_b2,
               params["logits_W"], params["logits_b"][None, :])

    out = pl.pallas_call(
        _gnn_body,
        out_shape=jax.ShapeDtypeStruct((_B * _N, _F), jnp.float32),
    )(time, p.reshape(_B * _N, _F), *weights)
    return out.reshape(_B, _N, _F)


# speedup vs baseline: 1181.3736x; 2.0590x over previous
import functools

import jax
import jax.numpy as jnp
import numpy as np
from jax.experimental import pallas as pl

_EPS = 1e-3
_B, _N, _F, _T, _H = 2, 384, 128, 8, 64
_INV_DEG = 1.0 / (_N - 1)
_RSQ = 1.0 / np.sqrt(1.0 + _EPS)


def _rows_per_batch(v):
    d = v.shape[1]
    return jnp.concatenate(
        [jnp.broadcast_to(v[i:i + 1], (_N, d)) for i in range(_B)], axis=0)


def _mean_of_others(g):
    d = g.shape[1]
    s = jnp.concatenate(
        [jnp.broadcast_to(jnp.sum(g[i * _N:(i + 1) * _N], axis=0,
                                  keepdims=True), (_N, d))
         for i in range(_B)], axis=0)
    return (s - g) * _INV_DEG


def _gnn_body(time_ref, p_ref, *refs):
    out_ref = refs[-1]
    (tm1, tm2, pre1, pre2, c1p1, c1p2, c1u1, c1u2,
     c2p1, c2p2, c2u1, c2u2, post1, post2, lg_w_ref, lg_b_ref) = (
        [refs[4 * i:4 * i + 4] for i in range(14)] + [refs[56], refs[57]])

    dot = functools.partial(jnp.dot, precision=jax.lax.Precision.HIGHEST,
                            preferred_element_type=jnp.float32)
    gelu = jax.nn.gelu

    def bn(v, ga_ref, be_ref, lo=None, hi=None):
        ga = ga_ref[...] if lo is None else ga_ref[:, lo:hi]
        be = be_ref[...] if lo is None else be_ref[:, lo:hi]
        return v * (ga * _RSQ) + be

    def layer(v, lp):
        ga, be, w, b = lp
        return gelu(dot(bn(v, ga, be), w[...]) + b[...])

    t = layer(layer(time_ref[...], tm1), tm2)

    x = layer(layer(p_ref[...], pre1), pre2)

    for (pl1, pl2, ul1, ul2) in ((c1p1, c1p2, c1u1, c1u2),
                                 (c2p1, c2p2, c2u1, c2u2)):
        pga, pbe, pw, pb = pl1
        tb = dot(bn(t, pga, pbe, _H, 2 * _H), pw[_H:, :]) + pb[...]
        g = gelu(dot(bn(x, pga, pbe, 0, _H), pw[:_H, :]) + _rows_per_batch(tb))
        g = layer(g, pl2)
        agg = _mean_of_others(g)
        uga, ube, uw, ub = ul1
        utb = dot(bn(t, uga, ube, 2 * _H, 3 * _H), uw[2 * _H:, :]) + ub[...]
        u = gelu(dot(bn(x, uga, ube, 0, _H), uw[:_H, :])
                 + dot(bn(agg, uga, ube, _H, 2 * _H), uw[_H:2 * _H, :])
                 + _rows_per_batch(utb))
        u = layer(u, ul2)
        x = x + u

    x = layer(layer(x, post1), post2)
    out_ref[...] = dot(x, lg_w_ref[...]) + lg_b_ref[...]


def kernel(p, time, params):
    def lp(layer_params):
        return (layer_params["gamma"][None, :], layer_params["beta"][None, :],
                layer_params["W"], layer_params["b"][None, :])

    weights = []
    for key in ("time_mlp", "pre", "c1_prep", "c1_upd",
                "c2_prep", "c2_upd", "post"):
        weights.extend(lp(params[key][0]))
        weights.extend(lp(params[key][1]))
    weights.append(params["logits_W"])
    weights.append(params["logits_b"][None, :])

    out = pl.pallas_call(
        _gnn_body,
        out_shape=jax.ShapeDtypeStruct((_B * _N, _F), jnp.float32),
    )(time, p.reshape(_B * _N, _F), *weights)
    return out.reshape(_B, _N, _F)


# speedup vs baseline: 2237.2588x; 1.8938x over previous
import functools

import jax
import jax.numpy as jnp
import numpy as np
from jax.experimental import pallas as pl

_EPS = 1e-3
_B, _N, _F, _T, _H = 2, 384, 128, 8, 64
_INV_DEG = 1.0 / (_N - 1)
_RSQ = 1.0 / np.sqrt(1.0 + _EPS)


def _rows_per_batch(v):
    d = v.shape[1]
    return jnp.concatenate(
        [jnp.broadcast_to(v[i:i + 1], (_N, d)) for i in range(_B)], axis=0)


def _mean_of_others(g):
    d = g.shape[1]
    s = jnp.concatenate(
        [jnp.broadcast_to(jnp.sum(g[i * _N:(i + 1) * _N], axis=0,
                                  keepdims=True), (_N, d))
         for i in range(_B)], axis=0)
    return (s - g) * _INV_DEG


def _gnn_body(time_ref, p_ref, *refs):
    out_ref = refs[-1]
    (tm1, tm2, pre1, pre2, c1p1, c1p2, c1u1, c1u2,
     c2p1, c2p2, c2u1, c2u2, post1, post2, lg_w_ref, lg_b_ref) = (
        [refs[4 * i:4 * i + 4] for i in range(14)] + [refs[56], refs[57]])

    dot = functools.partial(jnp.dot, precision=jax.lax.Precision.DEFAULT,
                            preferred_element_type=jnp.float32)
    gelu = jax.nn.gelu

    def bn(v, ga_ref, be_ref, lo=None, hi=None):
        ga = ga_ref[...] if lo is None else ga_ref[:, lo:hi]
        be = be_ref[...] if lo is None else be_ref[:, lo:hi]
        return v * (ga * _RSQ) + be

    def layer(v, lp):
        ga, be, w, b = lp
        return gelu(dot(bn(v, ga, be), w[...]) + b[...])

    t = layer(layer(time_ref[...], tm1), tm2)

    x = layer(layer(p_ref[...], pre1), pre2)

    for (pl1, pl2, ul1, ul2) in ((c1p1, c1p2, c1u1, c1u2),
                                 (c2p1, c2p2, c2u1, c2u2)):
        pga, pbe, pw, pb = pl1
        tb = dot(bn(t, pga, pbe, _H, 2 * _H), pw[_H:, :]) + pb[...]
        g = gelu(dot(bn(x, pga, pbe, 0, _H), pw[:_H, :]) + _rows_per_batch(tb))
        g = layer(g, pl2)
        agg = _mean_of_others(g)
        uga, ube, uw, ub = ul1
        utb = dot(bn(t, uga, ube, 2 * _H, 3 * _H), uw[2 * _H:, :]) + ub[...]
        u = gelu(dot(bn(x, uga, ube, 0, _H), uw[:_H, :])
                 + dot(bn(agg, uga, ube, _H, 2 * _H), uw[_H:2 * _H, :])
                 + _rows_per_batch(utb))
        u = layer(u, ul2)
        x = x + u

    x = layer(layer(x, post1), post2)
    out_ref[...] = dot(x, lg_w_ref[...]) + lg_b_ref[...]


def kernel(p, time, params):
    def lp(layer_params):
        return (layer_params["gamma"][None, :], layer_params["beta"][None, :],
                layer_params["W"], layer_params["b"][None, :])

    weights = []
    for key in ("time_mlp", "pre", "c1_prep", "c1_upd",
                "c2_prep", "c2_upd", "post"):
        weights.extend(lp(params[key][0]))
        weights.extend(lp(params[key][1]))
    weights.append(params["logits_W"])
    weights.append(params["logits_b"][None, :])

    out = pl.pallas_call(
        _gnn_body,
        out_shape=jax.ShapeDtypeStruct((_B * _N, _F), jnp.float32),
    )(time, p.reshape(_B * _N, _F), *weights)
    return out.reshape(_B, _N, _F)
